# baseline (speedup 1.0000x reference)
import functools

import jax
import jax.numpy as jnp
from jax import lax
from jax.experimental import pallas as pl
from jax.experimental.pallas import tpu as pltpu
from jax.experimental.pallas import tpu_sc as plsc

N_NODES = 100000
N_EDGES = 1600000
NPAD = 100352
HALF = NPAD // 2
HPAD = HALF + 128
ZRPT = HPAD // 16
ORPT = HALF // 16
LANE = 256
FIRE = 4
CPT = 392
MACROS = CPT // FIRE
UNROLL = 14
NBODY = MACROS // UNROLL
NCH = 16 * CPT
EPAD = NCH * LANE
BLK = 2048
GRID = NPAD // BLK

assert HPAD % 16 == 0 and (HPAD // 16) % 8 == 0


def _sc_propagate(tables, src2, dsts, zeros_tile):
    g_n = len(tables)
    mesh = plsc.VectorSubcoreMesh(core_axis_name="c", subcore_axis_name="s")

    @functools.partial(
        pl.kernel,
        out_type=[jax.ShapeDtypeStruct((NPAD, 16), jnp.float32)] * g_n,
        mesh=mesh,
        scratch_types=[
            pltpu.VMEM((4, FIRE, LANE), jnp.int32),
            pltpu.VMEM((4, FIRE, LANE), jnp.int32),
            pltpu.VMEM((3, FIRE, LANE, 16), jnp.float32),
            pltpu.VMEM_SHARED((HPAD, 16), jnp.float32),
            pltpu.SemaphoreType.DMA,
            pltpu.SemaphoreType.DMA,
            pltpu.SemaphoreType.DMA,
        ],
        compiler_params=pltpu.CompilerParams(use_tc_tiling_on_sc=False),
    )
    def k(*refs):
        table_hbms = refs[:g_n]
        src_hbm, dst_hbm, zero_hbm = refs[g_n:g_n + 3]
        out_hbms = refs[g_n + 3:2 * g_n + 3]
        sidx, didx, rows, acc, gsem, ssem, isem = refs[2 * g_n + 3:]
        c = lax.axis_index("c")
        s = lax.axis_index("s")

        for g in range(g_n):
            pltpu.sync_copy(zero_hbm, acc.at[pl.ds(s * ZRPT, ZRPT)])
            plsc.subcore_barrier()

            def body(i, carry, table_hbm=table_hbms[g]):
                base = s * CPT + i * (UNROLL * FIRE)

                def fire_gather(m):
                    bi, br = m % 4, m % 3
                    return [
                        pltpu.async_copy(
                            table_hbm.at[plsc.Indices(sidx.at[bi, j],
                                                      ignored_value=-1)],
                            rows.at[br, j], gsem)
                        for j in range(FIRE)
                    ]

                def prefetch_idx(m):
                    bi = m % 4
                    nrow = base + m * FIRE
                    pltpu.async_copy(src_hbm.at[c, pl.ds(nrow, FIRE)],
                                     sidx.at[bi], isem)
                    pltpu.async_copy(dst_hbm.at[c, pl.ds(nrow, FIRE)],
                                     didx.at[bi], isem)

                def drain_idx(m):
                    bi = m % 4
                    nrow = base + m * FIRE
                    pltpu.make_async_copy(src_hbm.at[c, pl.ds(nrow, FIRE)],
                                          sidx.at[bi], isem).wait()
                    pltpu.make_async_copy(dst_hbm.at[c, pl.ds(nrow, FIRE)],
                                          didx.at[bi], isem).wait()

                pltpu.sync_copy(src_hbm.at[c, pl.ds(base, FIRE)], sidx.at[0])
                pltpu.sync_copy(dst_hbm.at[c, pl.ds(base, FIRE)], didx.at[0])
                gpend = fire_gather(0)
                if UNROLL > 1:
                    prefetch_idx(1)
                spend = [None, None, None]
                for m in range(UNROLL):
                    br = m % 3
                    if spend[(m + 1) % 3] is not None:
                        for h in spend[(m + 1) % 3]:
                            h.wait()
                        spend[(m + 1) % 3] = None
                    for h in gpend:
                        h.wait()
                    if m + 1 < UNROLL:
                        drain_idx(m + 1)
                        gpend = fire_gather(m + 1)
                    if m + 2 < UNROLL:
                        prefetch_idx(m + 2)
                    spend[br] = [
                        pltpu.async_copy(
                            rows.at[br, j],
                            acc.at[plsc.Indices(didx.at[m % 4, j],
                                                ignored_value=-1)],
                            ssem, add=True)
                        for j in range(FIRE)
                    ]
                for p in spend:
                    if p is not None:
                        for h in p:
                            h.wait()
                return carry

            lax.fori_loop(0, NBODY, body, 0)
            plsc.subcore_barrier()
            pltpu.sync_copy(acc.at[pl.ds(s * ORPT, ORPT)],
                            out_hbms[g].at[pl.ds(c * HALF + s * ORPT, ORPT)])

    outs = k(*tables, src2, dsts, zeros_tile)
    return outs if isinstance(outs, (list, tuple)) else [outs]


def _sc_degree(srcs, dsts, srows_const, drows_const, zeros_tile):
    mesh = plsc.VectorSubcoreMesh(core_axis_name="c", subcore_axis_name="s")

    @functools.partial(
        pl.kernel,
        out_type=jax.ShapeDtypeStruct((NPAD, 16), jnp.float32),
        mesh=mesh,
        scratch_types=[
            pltpu.VMEM((2, FIRE, LANE), jnp.int32),
            pltpu.VMEM((2, FIRE, LANE), jnp.int32),
            pltpu.VMEM((LANE, 16), jnp.float32),
            pltpu.VMEM((LANE, 16), jnp.float32),
            pltpu.VMEM_SHARED((HPAD, 16), jnp.float32),
            pltpu.SemaphoreType.DMA,
            pltpu.SemaphoreType.DMA,
        ],
        compiler_params=pltpu.CompilerParams(use_tc_tiling_on_sc=False),
    )
    def k(src_hbm, dst_hbm, sconst_hbm, dconst_hbm, zero_hbm,
          out_hbm, sidx, didx, srows, drows, acc, ssem, isem):
        c = lax.axis_index("c")
        s = lax.axis_index("s")
        pltpu.sync_copy(sconst_hbm, srows)
        pltpu.sync_copy(dconst_hbm, drows)
        pltpu.sync_copy(zero_hbm, acc.at[pl.ds(s * ZRPT, ZRPT)])
        plsc.subcore_barrier()

        def body(i, carry):
            base = s * CPT + i * (UNROLL * FIRE)
            pltpu.sync_copy(src_hbm.at[c, pl.ds(base, FIRE)], sidx.at[0])
            pltpu.sync_copy(dst_hbm.at[c, pl.ds(base, FIRE)], didx.at[0])
            pend = [None, None]
            for m in range(UNROLL):
                b = m % 2
                if pend[1 - b] is not None:
                    for h in pend[1 - b]:
                        h.wait()
                    pend[1 - b] = None
                if m + 1 < UNROLL:
                    nrow = base + (m + 1) * FIRE
                    pltpu.async_copy(src_hbm.at[c, pl.ds(nrow, FIRE)],
                                     sidx.at[1 - b], isem)
                    pltpu.async_copy(dst_hbm.at[c, pl.ds(nrow, FIRE)],
                                     didx.at[1 - b], isem)
                sh = []
                for j in range(FIRE):
                    sh.append(pltpu.async_copy(
                        srows,
                        acc.at[plsc.Indices(sidx.at[b, j], ignored_value=-1)],
                        ssem, add=True))
                    sh.append(pltpu.async_copy(
                        drows,
                        acc.at[plsc.Indices(didx.at[b, j], ignored_value=-1)],
                        ssem, add=True))
                pend[b] = sh
                if m + 1 < UNROLL:
                    nrow = base + (m + 1) * FIRE
                    pltpu.make_async_copy(src_hbm.at[c, pl.ds(nrow, FIRE)],
                                          sidx.at[1 - b], isem).wait()
                    pltpu.make_async_copy(dst_hbm.at[c, pl.ds(nrow, FIRE)],
                                          didx.at[1 - b], isem).wait()
            for p in pend:
                if p is not None:
                    for h in p:
                        h.wait()
            return carry

        lax.fori_loop(0, NBODY, body, 0)
        plsc.subcore_barrier()
        pltpu.sync_copy(acc.at[pl.ds(s * ORPT, ORPT)],
                        out_hbm.at[pl.ds(c * HALF + s * ORPT, ORPT)])

    return k(srcs, dsts, srows_const, drows_const, zeros_tile)


def _mm_groups(xs, wref, n_out_groups):
    outs = []
    for go in range(n_out_groups):
        acc = None
        for g, xg in enumerate(xs):
            t = jnp.dot(xg, wref[g * 16:(g + 1) * 16, go * 16:(go + 1) * 16],
                        preferred_element_type=jnp.float32)
            acc = t if acc is None else acc + t
        outs.append(acc)
    return outs


def _tc_pre(deg, xpad):
    def body(deg_ref, x_ref, io_ref, ii_ref, t0_ref):
        deg_blk = deg_ref[...]
        io = lax.rsqrt(jnp.maximum(deg_blk[:, 0:1], 1.0))
        ii = lax.rsqrt(jnp.maximum(deg_blk[:, 1:2], 1.0))
        io_ref[...] = io
        ii_ref[...] = ii
        t0 = x_ref[...] * io
        lanes = lax.broadcasted_iota(jnp.int32, (BLK, 16), 1)
        t0_ref[...] = jnp.where(lanes == 0, t0, 0.0)

    return pl.pallas_call(
        body,
        grid=(GRID,),
        in_specs=[
            pl.BlockSpec((BLK, 16), lambda i: (i, 0)),
            pl.BlockSpec((BLK, 1), lambda i: (i, 0)),
        ],
        out_specs=[
            pl.BlockSpec((BLK, 1), lambda i: (i, 0)),
            pl.BlockSpec((BLK, 1), lambda i: (i, 0)),
            pl.BlockSpec((BLK, 16), lambda i: (i, 0)),
        ],
        out_shape=[
            jax.ShapeDtypeStruct((NPAD, 1), jnp.float32),
            jax.ShapeDtypeStruct((NPAD, 1), jnp.float32),
            jax.ShapeDtypeStruct((NPAD, 16), jnp.float32),
        ],
    )(deg, xpad)


def _tc_stage(aggs_in, inv_in, inv_out, w_cur, b_cur, w_next):
    gin = len(aggs_in)
    gh = (w_cur.shape[1] // 16) if w_cur is not None else gin
    gn = (w_next.shape[1] // 16) if w_next is not None else gh

    def body(*refs):
        it = iter(refs)
        p_refs = [next(it) for _ in range(gin)]
        ii_ref = next(it)
        io_ref = next(it)
        wc_ref = next(it) if w_cur is not None else None
        b_ref = next(it)
        wn_ref = next(it) if w_next is not None else None
        out_refs = [next(it) for _ in range(gn)]

        aggs = [p[...] for p in p_refs]
        hs = _mm_groups(aggs, wc_ref, gh) if wc_ref is not None else aggs
        ii = ii_ref[...]
        hs = [jnp.maximum(h * ii + b_ref[0:1, g * 16:(g + 1) * 16], 0.0)
              for g, h in enumerate(hs)]
        io = io_ref[...]
        us = [h * io for h in hs]
        outs = _mm_groups(us, wn_ref, gn) if wn_ref is not None else us
        for o_ref, o in zip(out_refs, outs):
            o_ref[...] = o

    in_specs = [pl.BlockSpec((BLK, 16), lambda i: (i, 0))
                for _ in range(gin)]
    in_specs.append(pl.BlockSpec((BLK, 1), lambda i: (i, 0)))
    in_specs.append(pl.BlockSpec((BLK, 1), lambda i: (i, 0)))
    args = list(aggs_in) + [inv_in, inv_out]
    if w_cur is not None:
        in_specs.append(pl.BlockSpec(w_cur.shape, lambda i: (0, 0)))
        args.append(w_cur)
    in_specs.append(pl.BlockSpec(b_cur.shape, lambda i: (0, 0)))
    args.append(b_cur)
    if w_next is not None:
        in_specs.append(pl.BlockSpec(w_next.shape, lambda i: (0, 0)))
        args.append(w_next)

    return pl.pallas_call(
        body,
        grid=(GRID,),
        in_specs=in_specs,
        out_specs=[pl.BlockSpec((BLK, 16), lambda i: (i, 0))] * gn,
        out_shape=[jax.ShapeDtypeStruct((NPAD, 16), jnp.float32)] * gn,
    )(*args)


def _tc_final(agg, inv_in, b_last):
    def body(p_ref, ii_ref, b_ref, o_ref):
        t = p_ref[:, 0:1] * ii_ref[...] + b_ref[0:1, 0:1]
        o_ref[...] = jax.nn.sigmoid(t)

    return pl.pallas_call(
        body,
        grid=(GRID,),
        in_specs=[
            pl.BlockSpec((BLK, 16), lambda i: (i, 0)),
            pl.BlockSpec((BLK, 1), lambda i: (i, 0)),
            pl.BlockSpec(b_last.shape, lambda i: (0, 0)),
        ],
        out_specs=pl.BlockSpec((BLK, 1), lambda i: (i, 0)),
        out_shape=jax.ShapeDtypeStruct((NPAD, 1), jnp.float32),
    )(agg, inv_in, b_last)


def kernel(x, edge_index, Ws, bs):
    n_layers = len(Ws)
    dims_in = [int(w.shape[0]) for w in Ws]
    dims_out = [int(w.shape[1]) for w in Ws]
    pre = [dims_in[k] <= dims_out[k] for k in range(n_layers)]

    wp, bp = [], []
    for k in range(n_layers):
        gi = -(-dims_in[k] // 16)
        go = -(-dims_out[k] // 16)
        w = jnp.zeros((gi * 16, go * 16), jnp.float32)
        wp.append(w.at[:dims_in[k], :dims_out[k]].set(Ws[k].astype(jnp.float32)))
        b = jnp.zeros((1, go * 16), jnp.float32)
        bp.append(b.at[0, :dims_out[k]].set(bs[k].astype(jnp.float32)))

    src = edge_index[0].astype(jnp.int32)
    dst = edge_index[1].astype(jnp.int32)
    pad_ids = jnp.full((EPAD - N_EDGES,), -1, dtype=jnp.int32)
    src_f = jnp.concatenate([src, pad_ids]).reshape(NCH, LANE)
    dst_f = jnp.concatenate([dst, pad_ids]).reshape(NCH, LANE)
    in0 = (dst_f >= 0) & (dst_f < HALF)
    in1 = dst_f >= HALF
    neg = jnp.full_like(dst_f, -1)
    dsts = jnp.stack([
        jnp.where(in0, dst_f, neg),
        jnp.where(in1, dst_f - HALF, neg),
    ])
    srcs_g = jnp.stack([
        jnp.where(in0, src_f, neg),
        jnp.where(in1, src_f, neg),
    ])
    sdeg = jnp.stack([
        jnp.where((src_f >= 0) & (src_f < HALF), src_f, neg),
        jnp.where(src_f >= HALF, src_f - HALF, neg),
    ])

    zeros_tile = jnp.zeros((ZRPT, 16), jnp.float32)
    srows = jnp.zeros((LANE, 16), jnp.float32).at[:, 0].set(1.0)
    drows = jnp.zeros((LANE, 16), jnp.float32).at[:, 1].set(1.0)

    deg = _sc_degree(sdeg, dsts, srows, drows, zeros_tile)
    xpad = jnp.zeros((NPAD, 1), jnp.float32).at[:N_NODES, 0].set(x[0])
    inv_out, inv_in, table0 = _tc_pre(deg, xpad)

    tables = [table0]
    out_col = None
    for k in range(n_layers):
        aggs = _sc_propagate(tables, srcs_g, dsts, zeros_tile)
        if k < n_layers - 1:
            w_cur = wp[k] if pre[k] else None
            w_next = wp[k + 1] if not pre[k + 1] else None
            tables = _tc_stage(aggs, inv_in, inv_out, w_cur, bp[k], w_next)
        else:
            out_col = _tc_final(aggs[0], inv_in, bp[k])

    return out_col[:N_NODES, 0].reshape(1, N_NODES)


# speedup vs baseline: 1.0082x; 1.0082x over previous
import functools

import jax
import jax.numpy as jnp
from jax import lax
from jax.experimental import pallas as pl
from jax.experimental.pallas import tpu as pltpu
from jax.experimental.pallas import tpu_sc as plsc

N_NODES = 100000
N_EDGES = 1600000
NPAD = 100352
HALF = NPAD // 2
HPAD = HALF + 128
ZRPT = HPAD // 16
ORPT = HALF // 16
LANE = 128
FIRE = 8
CPT = 784
MACROS = CPT // FIRE
UNROLL = 14
NBODY = MACROS // UNROLL
NCH = 16 * CPT
EPAD = NCH * LANE
BLK = 2048
GRID = NPAD // BLK

assert HPAD % 16 == 0 and (HPAD // 16) % 8 == 0


def _sc_propagate(tables, src2, dsts, zeros_tile):
    g_n = len(tables)
    mesh = plsc.VectorSubcoreMesh(core_axis_name="c", subcore_axis_name="s")

    @functools.partial(
        pl.kernel,
        out_type=[jax.ShapeDtypeStruct((NPAD, 16), jnp.float32)] * g_n,
        mesh=mesh,
        scratch_types=[
            pltpu.VMEM((4, FIRE, LANE), jnp.int32),
            pltpu.VMEM((4, FIRE, LANE), jnp.int32),
            pltpu.VMEM((3, FIRE, LANE, 16), jnp.float32),
            pltpu.VMEM_SHARED((HPAD, 16), jnp.float32),
            pltpu.SemaphoreType.DMA,
            pltpu.SemaphoreType.DMA,
            pltpu.SemaphoreType.DMA,
        ],
        compiler_params=pltpu.CompilerParams(use_tc_tiling_on_sc=False),
    )
    def k(*refs):
        table_hbms = refs[:g_n]
        src_hbm, dst_hbm, zero_hbm = refs[g_n:g_n + 3]
        out_hbms = refs[g_n + 3:2 * g_n + 3]
        sidx, didx, rows, acc, gsem, ssem, isem = refs[2 * g_n + 3:]
        c = lax.axis_index("c")
        s = lax.axis_index("s")

        for g in range(g_n):
            pltpu.sync_copy(zero_hbm, acc.at[pl.ds(s * ZRPT, ZRPT)])
            plsc.subcore_barrier()

            def body(i, carry, table_hbm=table_hbms[g]):
                base = s * CPT + i * (UNROLL * FIRE)

                def fire_gather(m):
                    bi, br = m % 4, m % 3
                    return [
                        pltpu.async_copy(
                            table_hbm.at[plsc.Indices(sidx.at[bi, j],
                                                      ignored_value=-1)],
                            rows.at[br, j], gsem)
                        for j in range(FIRE)
                    ]

                def prefetch_idx(m):
                    bi = m % 4
                    nrow = base + m * FIRE
                    pltpu.async_copy(src_hbm.at[c, pl.ds(nrow, FIRE)],
                                     sidx.at[bi], isem)
                    pltpu.async_copy(dst_hbm.at[c, pl.ds(nrow, FIRE)],
                                     didx.at[bi], isem)

                def drain_idx(m):
                    bi = m % 4
                    nrow = base + m * FIRE
                    pltpu.make_async_copy(src_hbm.at[c, pl.ds(nrow, FIRE)],
                                          sidx.at[bi], isem).wait()
                    pltpu.make_async_copy(dst_hbm.at[c, pl.ds(nrow, FIRE)],
                                          didx.at[bi], isem).wait()

                pltpu.sync_copy(src_hbm.at[c, pl.ds(base, FIRE)], sidx.at[0])
                pltpu.sync_copy(dst_hbm.at[c, pl.ds(base, FIRE)], didx.at[0])
                gpend = fire_gather(0)
                if UNROLL > 1:
                    prefetch_idx(1)
                spend = [None, None, None]
                for m in range(UNROLL):
                    br = m % 3
                    if spend[(m + 1) % 3] is not None:
                        for h in spend[(m + 1) % 3]:
                            h.wait()
                        spend[(m + 1) % 3] = None
                    for h in gpend:
                        h.wait()
                    if m + 1 < UNROLL:
                        drain_idx(m + 1)
                        gpend = fire_gather(m + 1)
                    if m + 2 < UNROLL:
                        prefetch_idx(m + 2)
                    spend[br] = [
                        pltpu.async_copy(
                            rows.at[br, j],
                            acc.at[plsc.Indices(didx.at[m % 4, j],
                                                ignored_value=-1)],
                            ssem, add=True)
                        for j in range(FIRE)
                    ]
                for p in spend:
                    if p is not None:
                        for h in p:
                            h.wait()
                return carry

            lax.fori_loop(0, NBODY, body, 0)
            plsc.subcore_barrier()
            pltpu.sync_copy(acc.at[pl.ds(s * ORPT, ORPT)],
                            out_hbms[g].at[pl.ds(c * HALF + s * ORPT, ORPT)])

    outs = k(*tables, src2, dsts, zeros_tile)
    return outs if isinstance(outs, (list, tuple)) else [outs]


def _sc_degree(srcs, dsts, srows_const, drows_const, zeros_tile):
    mesh = plsc.VectorSubcoreMesh(core_axis_name="c", subcore_axis_name="s")

    @functools.partial(
        pl.kernel,
        out_type=jax.ShapeDtypeStruct((NPAD, 16), jnp.float32),
        mesh=mesh,
        scratch_types=[
            pltpu.VMEM((2, FIRE, LANE), jnp.int32),
            pltpu.VMEM((2, FIRE, LANE), jnp.int32),
            pltpu.VMEM((LANE, 16), jnp.float32),
            pltpu.VMEM((LANE, 16), jnp.float32),
            pltpu.VMEM_SHARED((HPAD, 16), jnp.float32),
            pltpu.SemaphoreType.DMA,
            pltpu.SemaphoreType.DMA,
        ],
        compiler_params=pltpu.CompilerParams(use_tc_tiling_on_sc=False),
    )
    def k(src_hbm, dst_hbm, sconst_hbm, dconst_hbm, zero_hbm,
          out_hbm, sidx, didx, srows, drows, acc, ssem, isem):
        c = lax.axis_index("c")
        s = lax.axis_index("s")
        pltpu.sync_copy(sconst_hbm, srows)
        pltpu.sync_copy(dconst_hbm, drows)
        pltpu.sync_copy(zero_hbm, acc.at[pl.ds(s * ZRPT, ZRPT)])
        plsc.subcore_barrier()

        def body(i, carry):
            base = s * CPT + i * (UNROLL * FIRE)
            pltpu.sync_copy(src_hbm.at[c, pl.ds(base, FIRE)], sidx.at[0])
            pltpu.sync_copy(dst_hbm.at[c, pl.ds(base, FIRE)], didx.at[0])
            pend = [None, None]
            for m in range(UNROLL):
                b = m % 2
                if pend[1 - b] is not None:
                    for h in pend[1 - b]:
                        h.wait()
                    pend[1 - b] = None
                if m + 1 < UNROLL:
                    nrow = base + (m + 1) * FIRE
                    pltpu.async_copy(src_hbm.at[c, pl.ds(nrow, FIRE)],
                                     sidx.at[1 - b], isem)
                    pltpu.async_copy(dst_hbm.at[c, pl.ds(nrow, FIRE)],
                                     didx.at[1 - b], isem)
                sh = []
                for j in range(FIRE):
                    sh.append(pltpu.async_copy(
                        srows,
                        acc.at[plsc.Indices(sidx.at[b, j], ignored_value=-1)],
                        ssem, add=True))
                    sh.append(pltpu.async_copy(
                        drows,
                        acc.at[plsc.Indices(didx.at[b, j], ignored_value=-1)],
                        ssem, add=True))
                pend[b] = sh
                if m + 1 < UNROLL:
                    nrow = base + (m + 1) * FIRE
                    pltpu.make_async_copy(src_hbm.at[c, pl.ds(nrow, FIRE)],
                                          sidx.at[1 - b], isem).wait()
                    pltpu.make_async_copy(dst_hbm.at[c, pl.ds(nrow, FIRE)],
                                          didx.at[1 - b], isem).wait()
            for p in pend:
                if p is not None:
                    for h in p:
                        h.wait()
            return carry

        lax.fori_loop(0, NBODY, body, 0)
        plsc.subcore_barrier()
        pltpu.sync_copy(acc.at[pl.ds(s * ORPT, ORPT)],
                        out_hbm.at[pl.ds(c * HALF + s * ORPT, ORPT)])

    return k(srcs, dsts, srows_const, drows_const, zeros_tile)


def _mm_groups(xs, wref, n_out_groups):
    outs = []
    for go in range(n_out_groups):
        acc = None
        for g, xg in enumerate(xs):
            t = jnp.dot(xg, wref[g * 16:(g + 1) * 16, go * 16:(go + 1) * 16],
                        preferred_element_type=jnp.float32)
            acc = t if acc is None else acc + t
        outs.append(acc)
    return outs


def _tc_pre(deg, xpad):
    def body(deg_ref, x_ref, io_ref, ii_ref, t0_ref):
        deg_blk = deg_ref[...]
        io = lax.rsqrt(jnp.maximum(deg_blk[:, 0:1], 1.0))
        ii = lax.rsqrt(jnp.maximum(deg_blk[:, 1:2], 1.0))
        io_ref[...] = io
        ii_ref[...] = ii
        t0 = x_ref[...] * io
        lanes = lax.broadcasted_iota(jnp.int32, (BLK, 16), 1)
        t0_ref[...] = jnp.where(lanes == 0, t0, 0.0)

    return pl.pallas_call(
        body,
        grid=(GRID,),
        in_specs=[
            pl.BlockSpec((BLK, 16), lambda i: (i, 0)),
            pl.BlockSpec((BLK, 1), lambda i: (i, 0)),
        ],
        out_specs=[
            pl.BlockSpec((BLK, 1), lambda i: (i, 0)),
            pl.BlockSpec((BLK, 1), lambda i: (i, 0)),
            pl.BlockSpec((BLK, 16), lambda i: (i, 0)),
        ],
        out_shape=[
            jax.ShapeDtypeStruct((NPAD, 1), jnp.float32),
            jax.ShapeDtypeStruct((NPAD, 1), jnp.float32),
            jax.ShapeDtypeStruct((NPAD, 16), jnp.float32),
        ],
    )(deg, xpad)


def _tc_stage(aggs_in, inv_in, inv_out, w_cur, b_cur, w_next):
    gin = len(aggs_in)
    gh = (w_cur.shape[1] // 16) if w_cur is not None else gin
    gn = (w_next.shape[1] // 16) if w_next is not None else gh

    def body(*refs):
        it = iter(refs)
        p_refs = [next(it) for _ in range(gin)]
        ii_ref = next(it)
        io_ref = next(it)
        wc_ref = next(it) if w_cur is not None else None
        b_ref = next(it)
        wn_ref = next(it) if w_next is not None else None
        out_refs = [next(it) for _ in range(gn)]

        aggs = [p[...] for p in p_refs]
        hs = _mm_groups(aggs, wc_ref, gh) if wc_ref is not None else aggs
        ii = ii_ref[...]
        hs = [jnp.maximum(h * ii + b_ref[0:1, g * 16:(g + 1) * 16], 0.0)
              for g, h in enumerate(hs)]
        io = io_ref[...]
        us = [h * io for h in hs]
        outs = _mm_groups(us, wn_ref, gn) if wn_ref is not None else us
        for o_ref, o in zip(out_refs, outs):
            o_ref[...] = o

    in_specs = [pl.BlockSpec((BLK, 16), lambda i: (i, 0))
                for _ in range(gin)]
    in_specs.append(pl.BlockSpec((BLK, 1), lambda i: (i, 0)))
    in_specs.append(pl.BlockSpec((BLK, 1), lambda i: (i, 0)))
    args = list(aggs_in) + [inv_in, inv_out]
    if w_cur is not None:
        in_specs.append(pl.BlockSpec(w_cur.shape, lambda i: (0, 0)))
        args.append(w_cur)
    in_specs.append(pl.BlockSpec(b_cur.shape, lambda i: (0, 0)))
    args.append(b_cur)
    if w_next is not None:
        in_specs.append(pl.BlockSpec(w_next.shape, lambda i: (0, 0)))
        args.append(w_next)

    return pl.pallas_call(
        body,
        grid=(GRID,),
        in_specs=in_specs,
        out_specs=[pl.BlockSpec((BLK, 16), lambda i: (i, 0))] * gn,
        out_shape=[jax.ShapeDtypeStruct((NPAD, 16), jnp.float32)] * gn,
    )(*args)


def _tc_final(agg, inv_in, b_last):
    def body(p_ref, ii_ref, b_ref, o_ref):
        t = p_ref[:, 0:1] * ii_ref[...] + b_ref[0:1, 0:1]
        o_ref[...] = jax.nn.sigmoid(t)

    return pl.pallas_call(
        body,
        grid=(GRID,),
        in_specs=[
            pl.BlockSpec((BLK, 16), lambda i: (i, 0)),
            pl.BlockSpec((BLK, 1), lambda i: (i, 0)),
            pl.BlockSpec(b_last.shape, lambda i: (0, 0)),
        ],
        out_specs=pl.BlockSpec((BLK, 1), lambda i: (i, 0)),
        out_shape=jax.ShapeDtypeStruct((NPAD, 1), jnp.float32),
    )(agg, inv_in, b_last)


def kernel(x, edge_index, Ws, bs):
    n_layers = len(Ws)
    dims_in = [int(w.shape[0]) for w in Ws]
    dims_out = [int(w.shape[1]) for w in Ws]
    pre = [dims_in[k] <= dims_out[k] for k in range(n_layers)]

    wp, bp = [], []
    for k in range(n_layers):
        gi = -(-dims_in[k] // 16)
        go = -(-dims_out[k] // 16)
        w = jnp.zeros((gi * 16, go * 16), jnp.float32)
        wp.append(w.at[:dims_in[k], :dims_out[k]].set(Ws[k].astype(jnp.float32)))
        b = jnp.zeros((1, go * 16), jnp.float32)
        bp.append(b.at[0, :dims_out[k]].set(bs[k].astype(jnp.float32)))

    src = edge_index[0].astype(jnp.int32)
    dst = edge_index[1].astype(jnp.int32)
    pad_ids = jnp.full((EPAD - N_EDGES,), -1, dtype=jnp.int32)
    src_f = jnp.concatenate([src, pad_ids]).reshape(NCH, LANE)
    dst_f = jnp.concatenate([dst, pad_ids]).reshape(NCH, LANE)
    in0 = (dst_f >= 0) & (dst_f < HALF)
    in1 = dst_f >= HALF
    neg = jnp.full_like(dst_f, -1)
    dsts = jnp.stack([
        jnp.where(in0, dst_f, neg),
        jnp.where(in1, dst_f - HALF, neg),
    ])
    srcs_g = jnp.stack([
        jnp.where(in0, src_f, neg),
        jnp.where(in1, src_f, neg),
    ])
    sdeg = jnp.stack([
        jnp.where((src_f >= 0) & (src_f < HALF), src_f, neg),
        jnp.where(src_f >= HALF, src_f - HALF, neg),
    ])

    zeros_tile = jnp.zeros((ZRPT, 16), jnp.float32)
    srows = jnp.zeros((LANE, 16), jnp.float32).at[:, 0].set(1.0)
    drows = jnp.zeros((LANE, 16), jnp.float32).at[:, 1].set(1.0)

    deg = _sc_degree(sdeg, dsts, srows, drows, zeros_tile)
    xpad = jnp.zeros((NPAD, 1), jnp.float32).at[:N_NODES, 0].set(x[0])
    inv_out, inv_in, table0 = _tc_pre(deg, xpad)

    tables = [table0]
    out_col = None
    for k in range(n_layers):
        aggs = _sc_propagate(tables, srcs_g, dsts, zeros_tile)
        if k < n_layers - 1:
            w_cur = wp[k] if pre[k] else None
            w_next = wp[k + 1] if not pre[k + 1] else None
            tables = _tc_stage(aggs, inv_in, inv_out, w_cur, bp[k], w_next)
        else:
            out_col = _tc_final(aggs[0], inv_in, bp[k])

    return out_col[:N_NODES, 0].reshape(1, N_NODES)
